# Initial kernel scaffold; baseline (speedup 1.0000x reference)
import jax
import jax.numpy as jnp
from jax.experimental import pallas as pl


def kernel(user_id, pos_items, neg_items, user_table, item_table):
    raise NotImplementedError("write your pallas kernel here")


# speedup vs baseline: 5.6075x; 5.6075x over previous
import functools

import jax
import jax.numpy as jnp
from jax import lax
from jax.experimental import pallas as pl
from jax.experimental.pallas import tpu as pltpu
from jax.experimental.pallas import tpu_sc as plsc

D = 32
N_NEG = 100
NC = 2
NS = 16
NW = NC * NS
CB = 16
CROWS = CB * N_NEG


def _tower_body(bpw, uid_hbm, pid_hbm, nid_hbm, utab_hbm, itab_hbm,
                pos_out_hbm, neg_out_hbm,
                uid_v, pid_v, nid_v, urows_v, prows_v, nrows_v,
                posres_v, negres_v, sem):
    wid = lax.axis_index("s") * NC + lax.axis_index("c")
    nchunk = bpw // CB

    def chunk_body(c, _):
        b0 = wid * bpw + c * CB
        pltpu.sync_copy(uid_hbm.at[pl.ds(b0, CB)], uid_v)
        pltpu.sync_copy(pid_hbm.at[pl.ds(b0, CB)], pid_v)
        pltpu.sync_copy(nid_hbm.at[pl.ds(b0 * N_NEG, CROWS)], nid_v)
        cu = pltpu.async_copy(utab_hbm.at[uid_v], urows_v, sem)
        cp = pltpu.async_copy(itab_hbm.at[pid_v], prows_v, sem)
        cn = pltpu.async_copy(itab_hbm.at[nid_v], nrows_v, sem)
        cu.wait()
        cp.wait()
        cn.wait()

        lane = lax.iota(jnp.int32, 16)

        acc = jnp.zeros(16, jnp.float32)
        for j in range(CB):
            q0 = urows_v[j, pl.ds(0, 16)]
            q1 = urows_v[j, pl.ds(16, 16)]
            p0 = prows_v[j, pl.ds(0, 16)]
            p1 = prows_v[j, pl.ds(16, 16)]
            acc = jnp.where(lane == j, jnp.sum(p0 * q0 + p1 * q1), acc)
        posres_v[...] = acc

        def b_body(i, _):
            q0 = urows_v[i, pl.ds(0, 16)]
            q1 = urows_v[i, pl.ds(16, 16)]
            r_base = i * N_NEG
            for n0 in (0, 16, 32, 48, 64, 80, 84):
                acc = jnp.zeros(16, jnp.float32)
                for j in range(16):
                    r = r_base + n0 + j
                    e0 = nrows_v[r, pl.ds(0, 16)]
                    e1 = nrows_v[r, pl.ds(16, 16)]
                    acc = jnp.where(lane == j, jnp.sum(e0 * q0 + e1 * q1), acc)
                plsc.store_scatter(negres_v, [r_base + n0 + lane], acc)
            return 0

        lax.fori_loop(0, CB, b_body, 0)
        pltpu.sync_copy(posres_v, pos_out_hbm.at[pl.ds(b0, CB)])
        pltpu.sync_copy(negres_v, neg_out_hbm.at[pl.ds(b0 * N_NEG, CROWS)])
        return 0

    lax.fori_loop(0, nchunk, chunk_body, 0)


def kernel(user_id, pos_items, neg_items, user_table, item_table):
    b = user_id.shape[0]
    bpw = b // NW
    neg_flat = neg_items.reshape(-1)
    mesh = plsc.VectorSubcoreMesh(core_axis_name="c", subcore_axis_name="s")
    run = pl.kernel(
        functools.partial(_tower_body, bpw),
        out_type=(
            jax.ShapeDtypeStruct((b,), jnp.float32),
            jax.ShapeDtypeStruct((b * N_NEG,), jnp.float32),
        ),
        mesh=mesh,
        compiler_params=pltpu.CompilerParams(
            needs_layout_passes=False, use_tc_tiling_on_sc=False),
        scratch_types=[
            pltpu.VMEM((CB,), jnp.int32),
            pltpu.VMEM((CB,), jnp.int32),
            pltpu.VMEM((CROWS,), jnp.int32),
            pltpu.VMEM((CB, D), jnp.float32),
            pltpu.VMEM((CB, D), jnp.float32),
            pltpu.VMEM((CROWS, D), jnp.float32),
            pltpu.VMEM((CB,), jnp.float32),
            pltpu.VMEM((CROWS,), jnp.float32),
            pltpu.SemaphoreType.DMA,
        ],
    )
    pos_score, neg_score_flat = run(user_id, pos_items, neg_flat,
                                    user_table, item_table)
    return pos_score, neg_score_flat.reshape(b, N_NEG)
